# baseline (speedup 1.0000x reference)
import jax
import jax.numpy as jnp
from jax.experimental import pallas as pl
from jax.experimental.pallas import tpu as pltpu

N = 10000
BM = 80
NSTEP = N // BM


def _mlp(h, w1, b1, w2, b2, w3, b3):
    h = jnp.maximum(jnp.dot(h, w1, preferred_element_type=jnp.float32) + b1,
                    0.0)
    h = jnp.maximum(jnp.dot(h, w2, preferred_element_type=jnp.float32) + b2,
                    0.0)
    h = jnp.maximum(jnp.dot(h, w3, preferred_element_type=jnp.float32) + b3,
                    0.0)
    return h


def _layer0_body(eps_ref, a_ref, z_ref, zm_ref, w1_ref, b1_ref, w2_ref,
                 b2_ref, w3_ref, b3_ref, out_ref, outbf_ref, pool_ref,
                 abf_ref):
    a_bf = a_ref[...].astype(jnp.bfloat16)
    abf_ref[...] = a_bf
    agg = jnp.dot(a_bf, z_ref[...], preferred_element_type=jnp.float32)
    h = (1.0 + eps_ref[0, 0]) * zm_ref[...] + agg
    h = _mlp(h, w1_ref[...], b1_ref[...], w2_ref[...], b2_ref[...],
             w3_ref[...], b3_ref[...])
    out_ref[...] = h
    outbf_ref[...] = h.astype(jnp.bfloat16)

    @pl.when(pl.program_id(0) == 0)
    def _():
        pool_ref[...] = jnp.zeros_like(pool_ref)

    pool_ref[...] += jnp.sum(h, axis=0, keepdims=True)


def _gin_layer0(x, x_bf, a, Ws, bs, eps_l):
    f_in = x.shape[1]
    f_out = Ws[2].shape[1]
    return pl.pallas_call(
        _layer0_body,
        grid=(NSTEP,),
        in_specs=[
            pl.BlockSpec((1, 1), lambda m: (0, 0)),
            pl.BlockSpec((BM, N), lambda m: (m, 0)),
            pl.BlockSpec((N, f_in), lambda m: (0, 0)),
            pl.BlockSpec((BM, f_in), lambda m: (m, 0)),
            pl.BlockSpec((f_in, 64), lambda m: (0, 0)),
            pl.BlockSpec((1, 64), lambda m: (0, 0)),
            pl.BlockSpec((64, 64), lambda m: (0, 0)),
            pl.BlockSpec((1, 64), lambda m: (0, 0)),
            pl.BlockSpec((64, f_out), lambda m: (0, 0)),
            pl.BlockSpec((1, f_out), lambda m: (0, 0)),
        ],
        out_specs=[
            pl.BlockSpec((BM, f_out), lambda m: (m, 0)),
            pl.BlockSpec((BM, f_out), lambda m: (m, 0)),
            pl.BlockSpec((1, f_out), lambda m: (0, 0)),
            pl.BlockSpec((BM, N), lambda m: (m, 0)),
        ],
        out_shape=[
            jax.ShapeDtypeStruct((N, f_out), jnp.float32),
            jax.ShapeDtypeStruct((N, f_out), jnp.bfloat16),
            jax.ShapeDtypeStruct((1, f_out), jnp.float32),
            jax.ShapeDtypeStruct((N, N), jnp.bfloat16),
        ],
        compiler_params=pltpu.CompilerParams(
            dimension_semantics=("arbitrary",)),
    )(eps_l.reshape(1, 1), a, x_bf, x, Ws[0], bs[0].reshape(1, -1),
      Ws[1], bs[1].reshape(1, -1), Ws[2], bs[2].reshape(1, -1))


def _layers12_body(eps_ref, abf_ref, z1bf_ref, z1_ref, w1_ref, b1_ref,
                   w2_ref, b2_ref, w3_ref, b3_ref, pool2_ref, pool3_ref,
                   z2_ref, z2bf_ref):
    p = pl.program_id(0)
    m = pl.program_id(1)
    eps_l = jnp.where(p == 0, eps_ref[0, 0], eps_ref[0, 1])
    w1 = w1_ref[0]
    b1 = b1_ref[0]
    w2 = w2_ref[0]
    b2 = b2_ref[0]
    w3 = w3_ref[0]
    b3 = b3_ref[0]

    @pl.when(p == 0)
    def _():
        agg = jnp.dot(abf_ref[...], z1bf_ref[...],
                      preferred_element_type=jnp.float32)
        h = (1.0 + eps_l) * z1_ref[...] + agg
        h = _mlp(h, w1, b1, w2, b2, w3, b3)
        z2_ref[pl.ds(m * BM, BM), :] = h
        z2bf_ref[pl.ds(m * BM, BM), :] = h.astype(jnp.bfloat16)

        @pl.when(m == 0)
        def _():
            pool2_ref[...] = jnp.zeros_like(pool2_ref)

        pool2_ref[...] += jnp.sum(h, axis=0, keepdims=True)

    @pl.when(p == 1)
    def _():
        agg = jnp.dot(abf_ref[...], z2bf_ref[...],
                      preferred_element_type=jnp.float32)
        h = (1.0 + eps_l) * z2_ref[pl.ds(m * BM, BM), :] + agg
        h = _mlp(h, w1, b1, w2, b2, w3, b3)

        @pl.when(m == 0)
        def _():
            pool3_ref[...] = jnp.zeros_like(pool3_ref)

        pool3_ref[...] += jnp.sum(h, axis=0, keepdims=True)


def _gin_layers12(z1, z1_bf, a_bf, Ws1, bs1, Ws2, bs2, eps12):
    w1 = jnp.stack([Ws1[0], Ws2[0]])
    b1 = jnp.stack([bs1[0].reshape(1, -1), bs2[0].reshape(1, -1)])
    w2 = jnp.stack([Ws1[1], Ws2[1]])
    b2 = jnp.stack([bs1[1].reshape(1, -1), bs2[1].reshape(1, -1)])
    w3 = jnp.stack([Ws1[2], Ws2[2]])
    b3 = jnp.stack([bs1[2].reshape(1, -1), bs2[2].reshape(1, -1)])
    return pl.pallas_call(
        _layers12_body,
        grid=(2, NSTEP),
        in_specs=[
            pl.BlockSpec((1, 2), lambda p, m: (0, 0)),
            pl.BlockSpec((BM, N), lambda p, m: (m, 0)),
            pl.BlockSpec((N, 32), lambda p, m: (0, 0)),
            pl.BlockSpec((BM, 32), lambda p, m: (m, 0)),
            pl.BlockSpec((1, 32, 64), lambda p, m: (p, 0, 0)),
            pl.BlockSpec((1, 1, 64), lambda p, m: (p, 0, 0)),
            pl.BlockSpec((1, 64, 64), lambda p, m: (p, 0, 0)),
            pl.BlockSpec((1, 1, 64), lambda p, m: (p, 0, 0)),
            pl.BlockSpec((1, 64, 32), lambda p, m: (p, 0, 0)),
            pl.BlockSpec((1, 1, 32), lambda p, m: (p, 0, 0)),
        ],
        out_specs=[
            pl.BlockSpec((1, 32), lambda p, m: (0, 0)),
            pl.BlockSpec((1, 32), lambda p, m: (0, 0)),
        ],
        out_shape=[
            jax.ShapeDtypeStruct((1, 32), jnp.float32),
            jax.ShapeDtypeStruct((1, 32), jnp.float32),
        ],
        scratch_shapes=[
            pltpu.VMEM((N, 32), jnp.float32),
            pltpu.VMEM((N, 32), jnp.bfloat16),
        ],
        compiler_params=pltpu.CompilerParams(
            dimension_semantics=("arbitrary", "arbitrary")),
    )(eps12.reshape(1, 2), a_bf, z1_bf, z1, w1, b1, w2, b2, w3, b3)


def _head_body(x_ref, h1_ref, h2_ref, h3_ref, w1_ref, b1_ref, w2_ref, b2_ref,
               out_ref):
    px = jnp.sum(x_ref[...], axis=0, keepdims=True)
    res = jnp.concatenate([px, h1_ref[...], h2_ref[...], h3_ref[...]], axis=1)
    y = jnp.maximum(jnp.dot(res, w1_ref[...],
                            preferred_element_type=jnp.float32)
                    + b1_ref[...], 0.0)
    out_ref[...] = jnp.dot(y, w2_ref[...],
                           preferred_element_type=jnp.float32) + b2_ref[...]


def _head(x, pools, fc1_W, fc1_b, fc2_W, fc2_b):
    h1, h2, h3 = pools
    return pl.pallas_call(
        _head_body,
        out_shape=jax.ShapeDtypeStruct((1, 1), jnp.float32),
    )(x, h1, h2, h3, fc1_W, fc1_b.reshape(1, -1), fc2_W, fc2_b.reshape(1, -1))


def kernel(x, a, conv_Ws, conv_bs, eps, fc1_W, fc1_b, fc2_W, fc2_b):
    x_bf = x.astype(jnp.bfloat16)
    z1, z1_bf, p1, a_bf = _gin_layer0(x, x_bf, a, conv_Ws[0], conv_bs[0],
                                      eps[0])
    p2, p3 = _gin_layers12(z1, z1_bf, a_bf, conv_Ws[1], conv_bs[1],
                           conv_Ws[2], conv_bs[2], eps[1:3])
    return _head(x, (p1, p2, p3), fc1_W, fc1_b, fc2_W, fc2_b)


# speedup vs baseline: 1.5413x; 1.5413x over previous
import jax
import jax.numpy as jnp
from jax.experimental import pallas as pl
from jax.experimental.pallas import tpu as pltpu

N = 10000
BM = 400
NSTEP = N // BM


def _mlp(h, w1, b1, w2, b2, w3, b3):
    h = jnp.maximum(jnp.dot(h, w1, preferred_element_type=jnp.float32) + b1,
                    0.0)
    h = jnp.maximum(jnp.dot(h, w2, preferred_element_type=jnp.float32) + b2,
                    0.0)
    h = jnp.maximum(jnp.dot(h, w3, preferred_element_type=jnp.float32) + b3,
                    0.0)
    return h


def _layer0_body(eps_ref, a_ref, z_ref, zm_ref, w1_ref, b1_ref, w2_ref,
                 b2_ref, w3_ref, b3_ref, out_ref, outbf_ref, pool_ref,
                 abf_ref):
    a_bf = a_ref[...].astype(jnp.bfloat16)
    abf_ref[...] = a_bf
    agg = jnp.dot(a_bf, z_ref[...], preferred_element_type=jnp.float32)
    h = (1.0 + eps_ref[0, 0]) * zm_ref[...] + agg
    h = _mlp(h, w1_ref[...], b1_ref[...], w2_ref[...], b2_ref[...],
             w3_ref[...], b3_ref[...])
    out_ref[...] = h
    outbf_ref[...] = h.astype(jnp.bfloat16)

    @pl.when(pl.program_id(0) == 0)
    def _():
        pool_ref[...] = jnp.zeros_like(pool_ref)

    pool_ref[...] += jnp.sum(h, axis=0, keepdims=True)


def _gin_layer0(x, x_bf, a, Ws, bs, eps_l):
    f_in = x.shape[1]
    f_out = Ws[2].shape[1]
    return pl.pallas_call(
        _layer0_body,
        grid=(NSTEP,),
        in_specs=[
            pl.BlockSpec((1, 1), lambda m: (0, 0)),
            pl.BlockSpec((BM, N), lambda m: (m, 0)),
            pl.BlockSpec((N, f_in), lambda m: (0, 0)),
            pl.BlockSpec((BM, f_in), lambda m: (m, 0)),
            pl.BlockSpec((f_in, 64), lambda m: (0, 0)),
            pl.BlockSpec((1, 64), lambda m: (0, 0)),
            pl.BlockSpec((64, 64), lambda m: (0, 0)),
            pl.BlockSpec((1, 64), lambda m: (0, 0)),
            pl.BlockSpec((64, f_out), lambda m: (0, 0)),
            pl.BlockSpec((1, f_out), lambda m: (0, 0)),
        ],
        out_specs=[
            pl.BlockSpec((BM, f_out), lambda m: (m, 0)),
            pl.BlockSpec((BM, f_out), lambda m: (m, 0)),
            pl.BlockSpec((1, f_out), lambda m: (0, 0)),
            pl.BlockSpec((BM, N), lambda m: (m, 0)),
        ],
        out_shape=[
            jax.ShapeDtypeStruct((N, f_out), jnp.float32),
            jax.ShapeDtypeStruct((N, f_out), jnp.bfloat16),
            jax.ShapeDtypeStruct((1, f_out), jnp.float32),
            jax.ShapeDtypeStruct((N, N), jnp.bfloat16),
        ],
        compiler_params=pltpu.CompilerParams(
            dimension_semantics=("arbitrary",)),
    )(eps_l.reshape(1, 1), a, x_bf, x, Ws[0], bs[0].reshape(1, -1),
      Ws[1], bs[1].reshape(1, -1), Ws[2], bs[2].reshape(1, -1))


def _layers12_body(eps_ref, abf_ref, z1bf_ref, z1_ref, w1_ref, b1_ref,
                   w2_ref, b2_ref, w3_ref, b3_ref, pool2_ref, pool3_ref,
                   z2_ref, z2bf_ref):
    p = pl.program_id(0)
    m = pl.program_id(1)
    eps_l = jnp.where(p == 0, eps_ref[0, 0], eps_ref[0, 1])
    w1 = w1_ref[0]
    b1 = b1_ref[0]
    w2 = w2_ref[0]
    b2 = b2_ref[0]
    w3 = w3_ref[0]
    b3 = b3_ref[0]

    @pl.when(p == 0)
    def _():
        agg = jnp.dot(abf_ref[...], z1bf_ref[...],
                      preferred_element_type=jnp.float32)
        h = (1.0 + eps_l) * z1_ref[...] + agg
        h = _mlp(h, w1, b1, w2, b2, w3, b3)
        z2_ref[pl.ds(m * BM, BM), :] = h
        z2bf_ref[pl.ds(m * BM, BM), :] = h.astype(jnp.bfloat16)

        @pl.when(m == 0)
        def _():
            pool2_ref[...] = jnp.zeros_like(pool2_ref)

        pool2_ref[...] += jnp.sum(h, axis=0, keepdims=True)

    @pl.when(p == 1)
    def _():
        agg = jnp.dot(abf_ref[...], z2bf_ref[...],
                      preferred_element_type=jnp.float32)
        h = (1.0 + eps_l) * z2_ref[pl.ds(m * BM, BM), :] + agg
        h = _mlp(h, w1, b1, w2, b2, w3, b3)

        @pl.when(m == 0)
        def _():
            pool3_ref[...] = jnp.zeros_like(pool3_ref)

        pool3_ref[...] += jnp.sum(h, axis=0, keepdims=True)


def _gin_layers12(z1, z1_bf, a_bf, Ws1, bs1, Ws2, bs2, eps12):
    w1 = jnp.stack([Ws1[0], Ws2[0]])
    b1 = jnp.stack([bs1[0].reshape(1, -1), bs2[0].reshape(1, -1)])
    w2 = jnp.stack([Ws1[1], Ws2[1]])
    b2 = jnp.stack([bs1[1].reshape(1, -1), bs2[1].reshape(1, -1)])
    w3 = jnp.stack([Ws1[2], Ws2[2]])
    b3 = jnp.stack([bs1[2].reshape(1, -1), bs2[2].reshape(1, -1)])
    return pl.pallas_call(
        _layers12_body,
        grid=(2, NSTEP),
        in_specs=[
            pl.BlockSpec((1, 2), lambda p, m: (0, 0)),
            pl.BlockSpec((BM, N), lambda p, m: (m, 0)),
            pl.BlockSpec((N, 32), lambda p, m: (0, 0)),
            pl.BlockSpec((BM, 32), lambda p, m: (m, 0)),
            pl.BlockSpec((1, 32, 64), lambda p, m: (p, 0, 0)),
            pl.BlockSpec((1, 1, 64), lambda p, m: (p, 0, 0)),
            pl.BlockSpec((1, 64, 64), lambda p, m: (p, 0, 0)),
            pl.BlockSpec((1, 1, 64), lambda p, m: (p, 0, 0)),
            pl.BlockSpec((1, 64, 32), lambda p, m: (p, 0, 0)),
            pl.BlockSpec((1, 1, 32), lambda p, m: (p, 0, 0)),
        ],
        out_specs=[
            pl.BlockSpec((1, 32), lambda p, m: (0, 0)),
            pl.BlockSpec((1, 32), lambda p, m: (0, 0)),
        ],
        out_shape=[
            jax.ShapeDtypeStruct((1, 32), jnp.float32),
            jax.ShapeDtypeStruct((1, 32), jnp.float32),
        ],
        scratch_shapes=[
            pltpu.VMEM((N, 32), jnp.float32),
            pltpu.VMEM((N, 32), jnp.bfloat16),
        ],
        compiler_params=pltpu.CompilerParams(
            dimension_semantics=("arbitrary", "arbitrary")),
    )(eps12.reshape(1, 2), a_bf, z1_bf, z1, w1, b1, w2, b2, w3, b3)


def _head_body(x_ref, h1_ref, h2_ref, h3_ref, w1_ref, b1_ref, w2_ref, b2_ref,
               out_ref):
    px = jnp.sum(x_ref[...], axis=0, keepdims=True)
    res = jnp.concatenate([px, h1_ref[...], h2_ref[...], h3_ref[...]], axis=1)
    y = jnp.maximum(jnp.dot(res, w1_ref[...],
                            preferred_element_type=jnp.float32)
                    + b1_ref[...], 0.0)
    out_ref[...] = jnp.dot(y, w2_ref[...],
                           preferred_element_type=jnp.float32) + b2_ref[...]


def _head(x, pools, fc1_W, fc1_b, fc2_W, fc2_b):
    h1, h2, h3 = pools
    return pl.pallas_call(
        _head_body,
        out_shape=jax.ShapeDtypeStruct((1, 1), jnp.float32),
    )(x, h1, h2, h3, fc1_W, fc1_b.reshape(1, -1), fc2_W, fc2_b.reshape(1, -1))


def kernel(x, a, conv_Ws, conv_bs, eps, fc1_W, fc1_b, fc2_W, fc2_b):
    x_bf = x.astype(jnp.bfloat16)
    z1, z1_bf, p1, a_bf = _gin_layer0(x, x_bf, a, conv_Ws[0], conv_bs[0],
                                      eps[0])
    p2, p3 = _gin_layers12(z1, z1_bf, a_bf, conv_Ws[1], conv_bs[1],
                           conv_Ws[2], conv_bs[2], eps[1:3])
    return _head(x, (p1, p2, p3), fc1_W, fc1_b, fc2_W, fc2_b)
